# baseline (speedup 1.0000x reference)
import jax
import jax.numpy as jnp
from jax.experimental import pallas as pl
from jax.experimental.pallas import tpu as pltpu

BATCH = 1024
FEATS = 4096
BN = 512


NC = 4


def _matmul_body(x_ref, w_ref, b_ref, o_ref, x16_ref):
    @pl.when(pl.program_id(0) == 0)
    def _cast_x():
        x16_ref[...] = x_ref[...].astype(jnp.bfloat16)

    x16 = x16_ref[...]
    ch = BN // NC
    for c in range(NC):
        sl = pl.ds(c * ch, ch)
        wc = w_ref[sl, :].astype(jnp.bfloat16)
        acc = jax.lax.dot_general(
            x16, wc,
            dimension_numbers=(((1,), (1,)), ((), ())),
            preferred_element_type=jnp.float32,
        )
        o_ref[:, sl] = acc + b_ref[:, sl]


def kernel(x, weight, bias):
    bias2d = bias.reshape(1, FEATS)
    grid = (FEATS // BN,)
    return pl.pallas_call(
        _matmul_body,
        grid=grid,
        in_specs=[
            pl.BlockSpec((BATCH, FEATS), lambda j: (0, 0)),
            pl.BlockSpec((BN, FEATS), lambda j: (j, 0)),
            pl.BlockSpec((1, BN), lambda j: (0, j)),
        ],
        out_specs=pl.BlockSpec((BATCH, BN), lambda j: (0, j)),
        out_shape=jax.ShapeDtypeStruct((BATCH, FEATS), jnp.float32),
        scratch_shapes=[pltpu.VMEM((BATCH, FEATS), jnp.bfloat16)],
        compiler_params=pltpu.CompilerParams(
            dimension_semantics=("arbitrary",),
        ),
    )(x, weight, bias2d)


# speedup vs baseline: 1.6996x; 1.6996x over previous
import jax
import jax.numpy as jnp
from jax.experimental import pallas as pl
from jax.experimental.pallas import tpu as pltpu

BATCH = 1024
FEATS = 4096
BN = 512


def _matmul_body(x_ref, w_ref, b_ref, o_ref):
    acc = jax.lax.dot_general(
        x_ref[...], w_ref[...],
        dimension_numbers=(((1,), (1,)), ((), ())),
        preferred_element_type=jnp.float32,
        precision=jax.lax.Precision.DEFAULT,
    )
    o_ref[...] = acc + b_ref[...]


def kernel(x, weight, bias):
    bias2d = bias.reshape(1, FEATS)
    grid = (FEATS // BN,)
    return pl.pallas_call(
        _matmul_body,
        grid=grid,
        in_specs=[
            pl.BlockSpec((BATCH, FEATS), lambda j: (0, 0)),
            pl.BlockSpec((BN, FEATS), lambda j: (j, 0)),
            pl.BlockSpec((1, BN), lambda j: (0, j)),
        ],
        out_specs=pl.BlockSpec((BATCH, BN), lambda j: (0, j)),
        out_shape=jax.ShapeDtypeStruct((BATCH, FEATS), jnp.float32),
        compiler_params=pltpu.CompilerParams(
            dimension_semantics=("arbitrary",),
        ),
    )(x, weight, bias2d)
